# baseline (speedup 1.0000x reference)
import functools

import jax
import jax.numpy as jnp
from jax import lax
from jax.experimental import pallas as pl
from jax.experimental.pallas import tpu as pltpu
from jax.experimental.pallas import tpu_sc as plsc

_BINS = (-1.1503, -0.6745, -0.3186, 0.0, 0.3186, 0.6745, 1.1503)

_N = 32768
_D = 1024
_NW = 32
_CHUNK = _N // _NW
_L = 16


def _bucketize_body(x_hbm, out_hbm, y_v, r_v):
    c = lax.axis_index("c")
    s = lax.axis_index("s")
    wid = s * 2 + c
    base = wid * _CHUNK
    pltpu.sync_copy(x_hbm.at[pl.ds(base, _CHUNK), pl.ds(0, 1)], y_v)

    lane = lax.iota(jnp.int32, _L)
    zero = jnp.zeros((_L,), jnp.int32)
    for i in range(_CHUNK // _L):
        y = plsc.load_gather(y_v, [i * _L + lane, zero])
        r = jnp.zeros((_L,), jnp.int32)
        for b in _BINS:
            r = r + (y > jnp.float32(b)).astype(jnp.int32)
        r_v[pl.ds(i * _L, _L)] = r
    pltpu.sync_copy(r_v, out_hbm.at[pl.ds(base, _CHUNK)])


def kernel(x):
    xr = x.reshape(2 * _N, _D)
    mesh = plsc.VectorSubcoreMesh(core_axis_name="c", subcore_axis_name="s")
    run = functools.partial(
        pl.kernel,
        mesh=mesh,
        out_type=jax.ShapeDtypeStruct((_N,), jnp.int32),
        scratch_types=[
            pltpu.VMEM((_CHUNK, 1), jnp.float32),
            pltpu.VMEM((_CHUNK,), jnp.int32),
        ],
        compiler_params=pltpu.CompilerParams(
            use_tc_tiling_on_sc=False, needs_layout_passes=False
        ),
    )(_bucketize_body)
    return run(xr)


# speedup vs baseline: 7.2311x; 7.2311x over previous
import functools

import jax
import jax.numpy as jnp
from jax import lax
from jax.experimental import pallas as pl
from jax.experimental.pallas import tpu as pltpu
from jax.experimental.pallas import tpu_sc as plsc

_BINS = (-1.1503, -0.6745, -0.3186, 0.0, 0.3186, 0.6745, 1.1503)

_N = 32768
_D = 1024
_NW = 32
_CHUNK = _N // _NW
_L = 16
_SLAB = 256
_NSLAB = _CHUNK // _SLAB


def _bucketize_body(x_hbm, out_hbm, a_v, b_v, r_v, sem_a, sem_b):
    c = lax.axis_index("c")
    s = lax.axis_index("s")
    wid = s * 2 + c
    base = wid * _CHUNK

    bufs = (a_v, b_v)
    sems = (sem_a, sem_b)

    def start(k):
        return pltpu.async_copy(
            x_hbm.at[pl.ds(base + k * _SLAB, _SLAB), pl.ds(0, 128)],
            bufs[k % 2],
            sems[k % 2],
        )

    lane = lax.iota(jnp.int32, _L)
    zero = jnp.zeros((_L,), jnp.int32)

    cp = start(0)
    for k in range(_NSLAB):
        cp.wait()
        if k + 1 < _NSLAB:
            cp = start(k + 1)
        slab = bufs[k % 2]
        for g in range(_SLAB // _L):
            y = plsc.load_gather(slab, [g * _L + lane, zero])
            r = jnp.zeros((_L,), jnp.int32)
            for b in _BINS:
                r = r + (y > jnp.float32(b)).astype(jnp.int32)
            r_v[pl.ds(k * _SLAB + g * _L, _L)] = r

    pltpu.sync_copy(r_v, out_hbm.at[pl.ds(base, _CHUNK)])


def kernel(x):
    xr = x.reshape(2 * _N, _D)
    mesh = plsc.VectorSubcoreMesh(core_axis_name="c", subcore_axis_name="s")
    run = functools.partial(
        pl.kernel,
        mesh=mesh,
        out_type=jax.ShapeDtypeStruct((_N,), jnp.int32),
        scratch_types=[
            pltpu.VMEM((_SLAB, 128), jnp.float32),
            pltpu.VMEM((_SLAB, 128), jnp.float32),
            pltpu.VMEM((_CHUNK,), jnp.int32),
            pltpu.SemaphoreType.DMA,
            pltpu.SemaphoreType.DMA,
        ],
        compiler_params=pltpu.CompilerParams(
            use_tc_tiling_on_sc=True, needs_layout_passes=False
        ),
    )(_bucketize_body)
    return run(xr)
